# baseline (speedup 1.0000x reference)
import functools

import jax
import jax.numpy as jnp
from jax import lax
from jax.experimental import pallas as pl
from jax.experimental.pallas import tpu as pltpu
from jax.experimental.pallas import tpu_sc as plsc

E = 320000
V = 10000
D = 128
DW = D // 2
NC, NS = 2, 16
NW = NC * NS
EPW = E // NW
C = 80
NCHUNK = EPW // C


def _dot_chunk(rows1_v, rows2_v, out_v, obase):
    lane15 = lax.iota(jnp.int32, 16) == 15

    def body(e, _):
        acc1 = jnp.zeros((16,), jnp.float32)
        acc2 = jnp.zeros((16,), jnp.float32)
        for w in range(DW // 16):
            a = plsc.bitcast(rows1_v[e, pl.ds(16 * w, 16)], jnp.bfloat16)
            b = plsc.bitcast(rows2_v[e, pl.ds(16 * w, 16)], jnp.bfloat16)
            pa, pb = plsc.unpack(a * b, format=plsc.PackFormat.INTERLEAVED)
            acc1 = acc1 + pa
            acc2 = acc2 + pb
        csum = plsc.cumsum(acc1 + acc2)
        plsc.store_scatter(
            out_v, [jnp.full((16,), obase + e, jnp.int32)], csum, mask=lane15
        )
        return 0

    lax.fori_loop(0, C, body, 0, unroll=2)


def kernel(emb, edge_index):
    src = edge_index[0].astype(jnp.int32)
    dst = edge_index[1].astype(jnp.int32)
    emb_pk = jax.lax.bitcast_convert_type(
        emb.astype(jnp.bfloat16).reshape(V, DW, 2), jnp.int32
    )

    mesh = plsc.VectorSubcoreMesh(core_axis_name="c", subcore_axis_name="s")

    @functools.partial(
        pl.kernel,
        mesh=mesh,
        out_type=jax.ShapeDtypeStruct((E,), jnp.float32),
        compiler_params=pltpu.CompilerParams(
            needs_layout_passes=False, use_tc_tiling_on_sc=False
        ),
        scratch_types=[
            pltpu.VMEM((EPW,), jnp.int32),
            pltpu.VMEM((EPW,), jnp.int32),
            pltpu.VMEM((C, DW), jnp.int32),
            pltpu.VMEM((C, DW), jnp.int32),
            pltpu.VMEM((C, DW), jnp.int32),
            pltpu.VMEM((C, DW), jnp.int32),
            pltpu.VMEM((EPW,), jnp.float32),
            pltpu.SemaphoreType.DMA,
            pltpu.SemaphoreType.DMA,
            pltpu.SemaphoreType.DMA,
            pltpu.SemaphoreType.DMA,
        ],
    )
    def _k(emb_hbm, src_hbm, dst_hbm, out_hbm,
           idx1_all, idx2_all, r1a, r1b, r2a, r2b, out_all,
           s1a, s1b, s2a, s2b):
        wid = lax.axis_index("s") * NC + lax.axis_index("c")
        wbase = wid * EPW

        pltpu.sync_copy(src_hbm.at[pl.ds(wbase, EPW)], idx1_all)
        pltpu.sync_copy(dst_hbm.at[pl.ds(wbase, EPW)], idx2_all)

        def fire(i, r1, r2, s1, s2):
            off = pl.ds(i * C, C)
            pltpu.async_copy(emb_hbm.at[idx1_all.at[off]], r1, s1)
            pltpu.async_copy(emb_hbm.at[idx2_all.at[off]], r2, s2)

        def wait(r1, r2, s1, s2):
            off = pl.ds(0, C)
            pltpu.make_async_copy(emb_hbm.at[idx1_all.at[off]], r1, s1).wait()
            pltpu.make_async_copy(emb_hbm.at[idx2_all.at[off]], r2, s2).wait()

        fire(0, r1a, r2a, s1a, s2a)

        def body(k, _):
            i0 = 2 * k
            wait(r1a, r2a, s1a, s2a)
            fire(i0 + 1, r1b, r2b, s1b, s2b)
            _dot_chunk(r1a, r2a, out_all, i0 * C)
            wait(r1b, r2b, s1b, s2b)
            fire(i0 + 2, r1a, r2a, s1a, s2a)
            _dot_chunk(r1b, r2b, out_all, (i0 + 1) * C)
            return 0

        lax.fori_loop(0, (NCHUNK - 1) // 2, body, 0)

        wait(r1a, r2a, s1a, s2a)
        _dot_chunk(r1a, r2a, out_all, (NCHUNK - 1) * C)
        pltpu.sync_copy(out_all, out_hbm.at[pl.ds(wbase, EPW)])

    return _k(emb_pk, src, dst)


# speedup vs baseline: 1.1095x; 1.1095x over previous
import functools

import jax
import jax.numpy as jnp
from jax import lax
from jax.experimental import pallas as pl
from jax.experimental.pallas import tpu as pltpu
from jax.experimental.pallas import tpu_sc as plsc

E = 320000
V = 10000
D = 128
DW = D // 2
NC, NS = 2, 16
NW = NC * NS
EPW = E // NW
C = 80
NCHUNK = EPW // C


def _dot_chunk(rows1_v, rows2_v, out_v, obase):
    lane15 = lax.iota(jnp.int32, 16) == 15

    def body(e, _):
        acc1 = jnp.zeros((16,), jnp.float32)
        acc2 = jnp.zeros((16,), jnp.float32)
        for w in range(D // 32):
            a = rows1_v[e, pl.ds(32 * w, 32)]
            b = rows2_v[e, pl.ds(32 * w, 32)]
            pa, pb = plsc.unpack(a * b, format=plsc.PackFormat.INTERLEAVED)
            acc1 = acc1 + pa
            acc2 = acc2 + pb
        csum = plsc.cumsum(acc1 + acc2)
        plsc.store_scatter(
            out_v, [jnp.full((16,), obase + e, jnp.int32)], csum, mask=lane15
        )
        return 0

    lax.fori_loop(0, C, body, 0, unroll=2)


def kernel(emb, edge_index):
    src = edge_index[0].astype(jnp.int32)
    dst = edge_index[1].astype(jnp.int32)
    emb_bf = emb.astype(jnp.bfloat16)

    mesh = plsc.VectorSubcoreMesh(core_axis_name="c", subcore_axis_name="s")

    @functools.partial(
        pl.kernel,
        mesh=mesh,
        out_type=jax.ShapeDtypeStruct((E,), jnp.float32),
        compiler_params=pltpu.CompilerParams(
            needs_layout_passes=False, use_tc_tiling_on_sc=False
        ),
        scratch_types=[
            pltpu.VMEM((EPW,), jnp.int32),
            pltpu.VMEM((EPW,), jnp.int32),
            pltpu.VMEM((C, D), jnp.bfloat16),
            pltpu.VMEM((C, D), jnp.bfloat16),
            pltpu.VMEM((C, D), jnp.bfloat16),
            pltpu.VMEM((C, D), jnp.bfloat16),
            pltpu.VMEM((EPW,), jnp.float32),
            pltpu.SemaphoreType.DMA,
            pltpu.SemaphoreType.DMA,
            pltpu.SemaphoreType.DMA,
            pltpu.SemaphoreType.DMA,
        ],
    )
    def _k(emb_hbm, src_hbm, dst_hbm, out_hbm,
           idx1_all, idx2_all, r1a, r1b, r2a, r2b, out_all,
           s1a, s1b, s2a, s2b):
        wid = lax.axis_index("s") * NC + lax.axis_index("c")
        wbase = wid * EPW

        pltpu.sync_copy(src_hbm.at[pl.ds(wbase, EPW)], idx1_all)
        pltpu.sync_copy(dst_hbm.at[pl.ds(wbase, EPW)], idx2_all)

        def fire(i, r1, r2, s1, s2):
            off = pl.ds(i * C, C)
            pltpu.async_copy(emb_hbm.at[idx1_all.at[off]], r1, s1)
            pltpu.async_copy(emb_hbm.at[idx2_all.at[off]], r2, s2)

        def wait(r1, r2, s1, s2):
            off = pl.ds(0, C)
            pltpu.make_async_copy(emb_hbm.at[idx1_all.at[off]], r1, s1).wait()
            pltpu.make_async_copy(emb_hbm.at[idx2_all.at[off]], r2, s2).wait()

        fire(0, r1a, r2a, s1a, s2a)

        def body(k, _):
            i0 = 2 * k
            wait(r1a, r2a, s1a, s2a)
            fire(i0 + 1, r1b, r2b, s1b, s2b)
            _dot_chunk(r1a, r2a, out_all, i0 * C)
            wait(r1b, r2b, s1b, s2b)
            fire(i0 + 2, r1a, r2a, s1a, s2a)
            _dot_chunk(r1b, r2b, out_all, (i0 + 1) * C)
            return 0

        lax.fori_loop(0, (NCHUNK - 1) // 2, body, 0)

        wait(r1a, r2a, s1a, s2a)
        _dot_chunk(r1a, r2a, out_all, (NCHUNK - 1) * C)
        pltpu.sync_copy(out_all, out_hbm.at[pl.ds(wbase, EPW)])

    return _k(emb_bf, src, dst)


# speedup vs baseline: 1.1417x; 1.0290x over previous
import functools

import jax
import jax.numpy as jnp
from jax import lax
from jax.experimental import pallas as pl
from jax.experimental.pallas import tpu as pltpu
from jax.experimental.pallas import tpu_sc as plsc

E = 320000
V = 10000
D = 128
DW = D // 2
NC, NS = 2, 16
NW = NC * NS
EPW = E // NW
C = 80
NCHUNK = EPW // C


def _dot_chunk(rows1_v, rows2_v, out_v, obase):
    lane15 = lax.iota(jnp.int32, 16) == 15

    def body(e, _):
        acc1 = jnp.zeros((16,), jnp.float32)
        acc2 = jnp.zeros((16,), jnp.float32)
        for w in range(D // 32):
            a = rows1_v[e, pl.ds(32 * w, 32)]
            b = rows2_v[e, pl.ds(32 * w, 32)]
            pa, pb = plsc.unpack(a * b, format=plsc.PackFormat.INTERLEAVED)
            acc1 = acc1 + pa
            acc2 = acc2 + pb
        csum = plsc.cumsum(acc1 + acc2)
        plsc.store_scatter(
            out_v, [jnp.full((16,), obase + e, jnp.int32)], csum, mask=lane15
        )
        return 0

    lax.fori_loop(0, C, body, 0, unroll=4)


def kernel(emb, edge_index):
    src = edge_index[0].astype(jnp.int32)
    dst = edge_index[1].astype(jnp.int32)
    emb_bf = emb.astype(jnp.bfloat16)

    mesh = plsc.VectorSubcoreMesh(core_axis_name="c", subcore_axis_name="s")

    @functools.partial(
        pl.kernel,
        mesh=mesh,
        out_type=jax.ShapeDtypeStruct((E,), jnp.float32),
        compiler_params=pltpu.CompilerParams(
            needs_layout_passes=False, use_tc_tiling_on_sc=False
        ),
        scratch_types=[
            pltpu.VMEM((EPW,), jnp.int32),
            pltpu.VMEM((EPW,), jnp.int32),
            pltpu.VMEM((C, D), jnp.bfloat16),
            pltpu.VMEM((C, D), jnp.bfloat16),
            pltpu.VMEM((C, D), jnp.bfloat16),
            pltpu.VMEM((C, D), jnp.bfloat16),
            pltpu.VMEM((EPW,), jnp.float32),
            pltpu.SemaphoreType.DMA,
            pltpu.SemaphoreType.DMA,
            pltpu.SemaphoreType.DMA,
            pltpu.SemaphoreType.DMA,
        ],
    )
    def _k(emb_hbm, src_hbm, dst_hbm, out_hbm,
           idx1_all, idx2_all, r1a, r1b, r2a, r2b, out_all,
           s1a, s1b, s2a, s2b):
        wid = lax.axis_index("s") * NC + lax.axis_index("c")
        wbase = wid * EPW

        pltpu.sync_copy(src_hbm.at[pl.ds(wbase, EPW)], idx1_all)
        pltpu.sync_copy(dst_hbm.at[pl.ds(wbase, EPW)], idx2_all)

        def fire(i, r1, r2, s1, s2):
            off = pl.ds(i * C, C)
            pltpu.async_copy(emb_hbm.at[idx1_all.at[off]], r1, s1)
            pltpu.async_copy(emb_hbm.at[idx2_all.at[off]], r2, s2)

        def wait(r1, r2, s1, s2):
            off = pl.ds(0, C)
            pltpu.make_async_copy(emb_hbm.at[idx1_all.at[off]], r1, s1).wait()
            pltpu.make_async_copy(emb_hbm.at[idx2_all.at[off]], r2, s2).wait()

        fire(0, r1a, r2a, s1a, s2a)

        def body(k, _):
            i0 = 2 * k
            wait(r1a, r2a, s1a, s2a)
            fire(i0 + 1, r1b, r2b, s1b, s2b)
            _dot_chunk(r1a, r2a, out_all, i0 * C)
            wait(r1b, r2b, s1b, s2b)
            fire(i0 + 2, r1a, r2a, s1a, s2a)
            _dot_chunk(r1b, r2b, out_all, (i0 + 1) * C)
            return 0

        lax.fori_loop(0, (NCHUNK - 1) // 2, body, 0)

        wait(r1a, r2a, s1a, s2a)
        _dot_chunk(r1a, r2a, out_all, (NCHUNK - 1) * C)
        pltpu.sync_copy(out_all, out_hbm.at[pl.ds(wbase, EPW)])

    return _k(emb_bf, src, dst)


# speedup vs baseline: 1.1417x; 1.0000x over previous
import functools

import jax
import jax.numpy as jnp
from jax import lax
from jax.experimental import pallas as pl
from jax.experimental.pallas import tpu as pltpu
from jax.experimental.pallas import tpu_sc as plsc

E = 320000
V = 10000
D = 128
DW = D // 2
NC, NS = 2, 16
NW = NC * NS
EPW = E // NW
C = 200
PIECES = ((0, 104), (104, 96))
NCHUNK = EPW // C


def _dot_chunk(rows1_v, rows2_v, out_v, obase):
    lane15 = lax.iota(jnp.int32, 16) == 15

    def body(e, _):
        acc1 = jnp.zeros((16,), jnp.float32)
        acc2 = jnp.zeros((16,), jnp.float32)
        for w in range(D // 32):
            a = rows1_v[e, pl.ds(32 * w, 32)]
            b = rows2_v[e, pl.ds(32 * w, 32)]
            pa, pb = plsc.unpack(a * b, format=plsc.PackFormat.INTERLEAVED)
            acc1 = acc1 + pa
            acc2 = acc2 + pb
        csum = plsc.cumsum(acc1 + acc2)
        plsc.store_scatter(
            out_v, [jnp.full((16,), obase + e, jnp.int32)], csum, mask=lane15
        )
        return 0

    lax.fori_loop(0, C, body, 0, unroll=4)


def kernel(emb, edge_index):
    src = edge_index[0].astype(jnp.int32)
    dst = edge_index[1].astype(jnp.int32)
    emb_bf = emb.astype(jnp.bfloat16)

    mesh = plsc.VectorSubcoreMesh(core_axis_name="c", subcore_axis_name="s")

    @functools.partial(
        pl.kernel,
        mesh=mesh,
        out_type=jax.ShapeDtypeStruct((E,), jnp.float32),
        compiler_params=pltpu.CompilerParams(
            needs_layout_passes=False, use_tc_tiling_on_sc=False
        ),
        scratch_types=[
            pltpu.VMEM((EPW,), jnp.int32),
            pltpu.VMEM((EPW,), jnp.int32),
            pltpu.VMEM((C, D), jnp.bfloat16),
            pltpu.VMEM((C, D), jnp.bfloat16),
            pltpu.VMEM((C, D), jnp.bfloat16),
            pltpu.VMEM((C, D), jnp.bfloat16),
            pltpu.VMEM((EPW,), jnp.float32),
            pltpu.SemaphoreType.DMA,
            pltpu.SemaphoreType.DMA,
            pltpu.SemaphoreType.DMA,
            pltpu.SemaphoreType.DMA,
        ],
    )
    def _k(emb_hbm, src_hbm, dst_hbm, out_hbm,
           idx1_all, idx2_all, r1a, r1b, r2a, r2b, out_all,
           s1a, s1b, s2a, s2b):
        wid = lax.axis_index("s") * NC + lax.axis_index("c")
        wbase = wid * EPW

        pltpu.sync_copy(src_hbm.at[pl.ds(wbase, EPW)], idx1_all)
        pltpu.sync_copy(dst_hbm.at[pl.ds(wbase, EPW)], idx2_all)

        def fire(i, r1, r2, s1, s2):
            for po, pn in PIECES:
                off = pl.ds(i * C + po, pn)
                dst = pl.ds(po, pn)
                pltpu.async_copy(emb_hbm.at[idx1_all.at[off]], r1.at[dst], s1)
                pltpu.async_copy(emb_hbm.at[idx2_all.at[off]], r2.at[dst], s2)

        def wait(r1, r2, s1, s2):
            for po, pn in PIECES:
                off = pl.ds(po, pn)
                dst = pl.ds(po, pn)
                pltpu.make_async_copy(
                    emb_hbm.at[idx1_all.at[off]], r1.at[dst], s1).wait()
                pltpu.make_async_copy(
                    emb_hbm.at[idx2_all.at[off]], r2.at[dst], s2).wait()

        fire(0, r1a, r2a, s1a, s2a)

        def body(k, _):
            i0 = 2 * k
            wait(r1a, r2a, s1a, s2a)
            fire(i0 + 1, r1b, r2b, s1b, s2b)
            _dot_chunk(r1a, r2a, out_all, i0 * C)
            wait(r1b, r2b, s1b, s2b)

            @pl.when(i0 + 2 < NCHUNK)
            def _():
                fire(i0 + 2, r1a, r2a, s1a, s2a)

            _dot_chunk(r1b, r2b, out_all, (i0 + 1) * C)
            return 0

        lax.fori_loop(0, NCHUNK // 2, body, 0)

        pltpu.sync_copy(out_all, out_hbm.at[pl.ds(wbase, EPW)])

    return _k(emb_bf, src, dst)
